# baseline (speedup 1.0000x reference)
import functools

import jax
import jax.numpy as jnp
from jax import lax
from jax.experimental import pallas as pl
from jax.experimental.pallas import tpu as pltpu
from jax.experimental.pallas import tpu_sc as plsc

N = 10240
E = 163840
D = 256
H = 256
G = 64
C = 10
GF = 16

F = 128
NC = 2
NS = 16
CH = 128
EPW = E // NS
NCHG = EPW // CH
RPS = N // NS


def _sc_agg(h2, srcm, dstm, zeros):
    mesh = plsc.VectorSubcoreMesh(core_axis_name="c", subcore_axis_name="s")

    @functools.partial(
        pl.kernel,
        out_type=jax.ShapeDtypeStruct((NC * N, F), jnp.float32),
        mesh=mesh,
        scratch_types=[
            pltpu.VMEM((8, CH), jnp.int32),
            pltpu.VMEM((8, CH), jnp.int32),
            pltpu.VMEM((CH, F), jnp.float32),
            pltpu.VMEM((CH, F), jnp.float32),
            pltpu.SemaphoreType.DMA,
            pltpu.SemaphoreType.DMA,
            pltpu.VMEM_SHARED((N, F), jnp.float32),
        ],
    )
    def k(h2_hbm, srcm_hbm, dstm_hbm, zeros_hbm,
          out_hbm, srcv, dstv, rows0, rows1, sem0, sem1, acc):
        c = lax.axis_index("c")
        s = lax.axis_index("s")
        GS = 8

        pltpu.sync_copy(zeros_hbm.at[pl.ds(s * RPS, RPS)],
                        acc.at[pl.ds(s * RPS, RPS)])
        plsc.subcore_barrier()

        gbase = (c * NS + s) * NCHG
        dbase = s * NCHG

        HC = CH // 2

        def start(j, rows, sem):
            pltpu.async_copy(h2_hbm.at[srcv.at[j, pl.ds(0, HC)]],
                             rows.at[pl.ds(0, HC)], sem)
            pltpu.async_copy(h2_hbm.at[srcv.at[j, pl.ds(HC, HC)]],
                             rows.at[pl.ds(HC, HC)], sem)

        def wait(j, rows, sem):
            pltpu.make_async_copy(h2_hbm.at[srcv.at[j, pl.ds(0, HC)]],
                                  rows.at[pl.ds(0, HC)], sem).wait()
            pltpu.make_async_copy(h2_hbm.at[srcv.at[j, pl.ds(HC, HC)]],
                                  rows.at[pl.ds(HC, HC)], sem).wait()

        @pl.loop(0, NCHG // GS)
        def _(g):
            pltpu.sync_copy(srcm_hbm.at[pl.ds(gbase + g * GS, GS)], srcv)
            pltpu.sync_copy(dstm_hbm.at[pl.ds(dbase + g * GS, GS)], dstv)
            start(0, rows0, sem0)

            @pl.loop(0, GS, step=2)
            def _(j):
                start(j + 1, rows1, sem1)
                wait(j, rows0, sem0)
                pltpu.sync_copy(rows0, acc.at[dstv.at[j]], add=True)

                @pl.when(j + 2 < GS)
                def _():
                    start(j + 2, rows0, sem0)

                wait(j + 1, rows1, sem1)
                pltpu.sync_copy(rows1, acc.at[dstv.at[j + 1]], add=True)

        plsc.subcore_barrier()
        pltpu.sync_copy(acc.at[pl.ds(s * RPS, RPS)],
                        out_hbm.at[pl.ds(c * N + s * RPS, RPS)])

    return k(h2, srcm, dstm, zeros)


def _sc_count(dstm, zeros, onesf):
    mesh = plsc.VectorSubcoreMesh(core_axis_name="c", subcore_axis_name="s")
    nchw = NCHG // 2

    @functools.partial(
        pl.kernel,
        out_type=jax.ShapeDtypeStruct((NC * N, F), jnp.float32),
        mesh=mesh,
        scratch_types=[
            pltpu.VMEM((nchw, CH), jnp.int32),
            pltpu.VMEM((CH, F), jnp.float32),
            pltpu.VMEM_SHARED((N, F), jnp.float32),
        ],
    )
    def k(dstm_hbm, zeros_hbm, ones_hbm, cnt_hbm, dstv, ones_v, acc_c):
        c = lax.axis_index("c")
        s = lax.axis_index("s")

        pltpu.sync_copy(zeros_hbm.at[pl.ds(s * RPS, RPS)],
                        acc_c.at[pl.ds(s * RPS, RPS)])
        base = (c * NS + s) * nchw
        pltpu.sync_copy(dstm_hbm.at[pl.ds(base, nchw)], dstv)
        pltpu.sync_copy(ones_hbm, ones_v)
        plsc.subcore_barrier()

        @pl.loop(0, nchw)
        def _(j):
            pltpu.sync_copy(ones_v, acc_c.at[dstv.at[j]], add=True)

        plsc.subcore_barrier()
        pltpu.sync_copy(acc_c.at[pl.ds(s * RPS, RPS)],
                        cnt_hbm.at[pl.ds(c * N + s * RPS, RPS)])

    return k(dstm, zeros, onesf)


def _tc_layer(agg3, h3, cnt2, wlt, wrt, bl2, gg2, bb2):
    R = 512

    def body(agg_ref, h_ref, cnt_ref, wl_ref, wr_ref, b_ref, g_ref, be_ref,
             o_ref):
        agg = jnp.concatenate([agg_ref[0], agg_ref[1]], axis=1)
        h = jnp.concatenate([h_ref[0], h_ref[1]], axis=1)
        cnt = cnt_ref[0] + cnt_ref[1]
        recip = 1.0 / jnp.maximum(cnt, 1.0)
        z = (jnp.dot(agg * recip, wl_ref[...],
                     preferred_element_type=jnp.float32,
                     precision=lax.Precision.HIGHEST)
             + jnp.dot(h, wr_ref[...],
                       preferred_element_type=jnp.float32,
                       precision=lax.Precision.HIGHEST)
             + b_ref[...])
        mu = jnp.mean(z, axis=1, keepdims=True)
        zc = z - mu
        var = jnp.mean(zc * zc, axis=1, keepdims=True)
        hn = zc / jnp.sqrt(var + 1e-5) * g_ref[...] + be_ref[...]
        hn = jnp.maximum(hn, 0.0)
        o_ref[0] = hn[:, :F]
        o_ref[1] = hn[:, F:]

    return pl.pallas_call(
        body,
        grid=(N // R,),
        in_specs=[
            pl.BlockSpec((NC, R, F), lambda i: (0, i, 0)),
            pl.BlockSpec((NC, R, F), lambda i: (0, i, 0)),
            pl.BlockSpec((NC, R, 1), lambda i: (0, i, 0)),
            pl.BlockSpec((H, H), lambda i: (0, 0)),
            pl.BlockSpec((H, H), lambda i: (0, 0)),
            pl.BlockSpec((1, H), lambda i: (0, 0)),
            pl.BlockSpec((1, H), lambda i: (0, 0)),
            pl.BlockSpec((1, H), lambda i: (0, 0)),
        ],
        out_specs=pl.BlockSpec((NC, R, F), lambda i: (0, i, 0)),
        out_shape=jax.ShapeDtypeStruct((NC, N, F), jnp.float32),
    )(agg3, h3, cnt2, wlt, wrt, bl2, gg2, bb2)


def _tc_pool(h3, batch2, gfeat, wm1t, bm12, wm2t, bm22):
    K = 256
    nblk = N // K

    def body(h_ref, b_ref, gf_ref, w1_ref, b1_ref, w2_ref, b2_ref, o_ref,
             sum_s, max_s, cnt_s):
        i = pl.program_id(0)

        @pl.when(i == 0)
        def _():
            sum_s[...] = jnp.zeros_like(sum_s)
            max_s[...] = jnp.full_like(max_s, -3.4e38)
            cnt_s[...] = jnp.zeros_like(cnt_s)

        h = jnp.concatenate([h_ref[0], h_ref[1]], axis=1)
        b = b_ref[...]
        glo = jnp.min(b)
        ghi = jnp.max(b)

        def gbody(g, carry):
            mrow = b == g
            csum = jnp.sum(jnp.where(mrow, h, 0.0), axis=0, keepdims=True)
            cmax = jnp.max(jnp.where(mrow, h, -3.4e38), axis=0, keepdims=True)
            ccnt = jnp.sum(mrow.astype(jnp.float32))
            sum_s[pl.ds(g, 1), :] += csum
            max_s[pl.ds(g, 1), :] = jnp.maximum(max_s[pl.ds(g, 1), :], cmax)
            cnt_s[pl.ds(g, 1), :] += ccnt
            return carry

        lax.fori_loop(glo, ghi + 1, gbody, 0)

        @pl.when(i == nblk - 1)
        def _():
            cnt = cnt_s[:, 0:1]
            mean = sum_s[...] / jnp.maximum(cnt, 1.0)
            mx = jnp.where(cnt > 0.0, max_s[...], 0.0)
            gcat = jnp.concatenate([mean, mx, gf_ref[...]], axis=1)
            hm = jnp.maximum(
                jnp.dot(gcat, w1_ref[...],
                        preferred_element_type=jnp.float32,
                        precision=lax.Precision.HIGHEST) + b1_ref[...], 0.0)
            o_ref[...] = jnp.dot(hm, w2_ref[...],
                                 preferred_element_type=jnp.float32,
                                 precision=lax.Precision.HIGHEST) + b2_ref[...]

    return pl.pallas_call(
        body,
        grid=(nblk,),
        in_specs=[
            pl.BlockSpec((NC, K, F), lambda i: (0, i, 0)),
            pl.BlockSpec((K, 1), lambda i: (i, 0)),
            pl.BlockSpec((G, GF), lambda i: (0, 0)),
            pl.BlockSpec((2 * H + GF, H), lambda i: (0, 0)),
            pl.BlockSpec((1, H), lambda i: (0, 0)),
            pl.BlockSpec((H, C), lambda i: (0, 0)),
            pl.BlockSpec((1, C), lambda i: (0, 0)),
        ],
        out_specs=pl.BlockSpec((G, C), lambda i: (0, 0)),
        out_shape=jax.ShapeDtypeStruct((G, C), jnp.float32),
        scratch_shapes=[
            pltpu.VMEM((G, H), jnp.float32),
            pltpu.VMEM((G, H), jnp.float32),
            pltpu.VMEM((G, H), jnp.float32),
        ],
    )(h3, batch2, gfeat, wm1t, bm12, wm2t, bm22)


def kernel(x, edge_index, batch, ptr, root_idx, gfeat, W_l0, b_l0, W_r0, g0,
           be0, W_l1, b_l1, W_r1, g1, be1, W_l2, b_l2, W_r2, g2, be2, Wm1,
           bm1, Wm2, bm2):
    src = edge_index[0]
    dst = edge_index[1]
    src_r = src.reshape(NS * NCHG, CH)
    srcm = jnp.concatenate([src_r, src_r + N], axis=0)
    dstm = dst.reshape(NS * NCHG, CH)

    zeros = jnp.zeros((N, F), jnp.float32)
    onesf = jnp.ones((CH, F), jnp.float32)

    cntf = _sc_count(dstm, zeros, onesf)
    cnt3 = cntf[:, 0:1].reshape(NC, N, 1)

    h3 = x.reshape(N, NC, F).transpose(1, 0, 2)
    params = [(W_l0, b_l0, W_r0, g0, be0),
              (W_l1, b_l1, W_r1, g1, be1),
              (W_l2, b_l2, W_r2, g2, be2)]
    for wl, bl, wr, gg, bb in params:
        aggf = _sc_agg(h3.reshape(NC * N, F), srcm, dstm, zeros)
        h3 = _tc_layer(aggf.reshape(NC, N, F), h3, cnt3, wl.T, wr.T,
                       bl.reshape(1, H), gg.reshape(1, H), bb.reshape(1, H))

    return _tc_pool(h3, batch.reshape(N, 1), gfeat, Wm1.T,
                    bm1.reshape(1, H), Wm2.T, bm2.reshape(1, C))


# speedup vs baseline: 1.0589x; 1.0589x over previous
import dataclasses
import functools

import jax
import jax.numpy as jnp
from jax import lax
from jax.experimental import pallas as pl
from jax.experimental.pallas import tpu as pltpu
from jax.experimental.pallas import tpu_sc as plsc

N = 10240
E = 163840
D = 256
H = 256
G = 64
C = 10
GF = 16

F = 128
NC = 2
NS = 16
CH = 128
EPW = E // NS
NCHG = EPW // CH
RPS = N // NS


def _sc_agg(h2, srcm, dstm, zeros):
    mesh = plsc.VectorSubcoreMesh(core_axis_name="c", subcore_axis_name="s")

    @functools.partial(
        pl.kernel,
        out_type=jax.ShapeDtypeStruct((NC * N, F), jnp.float32),
        mesh=mesh,
        scratch_types=[
            pltpu.VMEM((8, CH), jnp.int32),
            pltpu.VMEM((8, CH), jnp.int32),
            pltpu.VMEM((CH, F), jnp.float32),
            pltpu.VMEM((CH, F), jnp.float32),
            pltpu.SemaphoreType.DMA,
            pltpu.SemaphoreType.DMA,
            pltpu.VMEM_SHARED((N, F), jnp.float32),
        ],
    )
    def k(h2_hbm, srcm_hbm, dstm_hbm, zeros_hbm,
          out_hbm, srcv, dstv, rows0, rows1, sem0, sem1, acc):
        c = lax.axis_index("c")
        s = lax.axis_index("s")
        GS = 8

        pltpu.sync_copy(zeros_hbm.at[pl.ds(s * RPS, RPS)],
                        acc.at[pl.ds(s * RPS, RPS)])
        plsc.subcore_barrier()

        gbase = (c * NS + s) * NCHG
        dbase = s * NCHG

        def start(j, rows, sem):
            pltpu.async_copy(h2_hbm.at[srcv.at[j]], rows, sem)

        def wait(j, rows, sem):
            pltpu.make_async_copy(h2_hbm.at[srcv.at[j]], rows, sem).wait()

        @pl.loop(0, NCHG // GS)
        def _(g):
            pltpu.sync_copy(srcm_hbm.at[pl.ds(gbase + g * GS, GS)], srcv)
            pltpu.sync_copy(dstm_hbm.at[pl.ds(dbase + g * GS, GS)], dstv)
            start(0, rows0, sem0)

            @pl.loop(0, GS, step=2)
            def _(j):
                start(j + 1, rows1, sem1)
                wait(j, rows0, sem0)
                pltpu.sync_copy(rows0, acc.at[dstv.at[j]], add=True)

                @pl.when(j + 2 < GS)
                def _():
                    start(j + 2, rows0, sem0)

                wait(j + 1, rows1, sem1)
                pltpu.sync_copy(rows1, acc.at[dstv.at[j + 1]], add=True)

        plsc.subcore_barrier()
        pltpu.sync_copy(acc.at[pl.ds(s * RPS, RPS)],
                        out_hbm.at[pl.ds(c * N + s * RPS, RPS)])

    return k(h2, srcm, dstm, zeros)


def _sc_count(dstf):
    mesh = plsc.VectorSubcoreMesh(core_axis_name="c", subcore_axis_name="s")
    epw = E // (NC * NS)

    @functools.partial(
        pl.kernel,
        out_type=jax.ShapeDtypeStruct((NC * N,), jnp.float32),
        mesh=mesh,
        scratch_types=[
            pltpu.VMEM((epw,), jnp.int32),
            pltpu.VMEM((N,), jnp.float32),
            pltpu.VMEM((RPS,), jnp.float32),
            pltpu.VMEM((RPS,), jnp.float32),
            pltpu.VMEM_SHARED((NS, N), jnp.float32),
        ],
        compiler_params=dataclasses.replace(pltpu.CompilerParams(),
                                            needs_layout_passes=False),
    )
    def k(dst_hbm, cnt_hbm, dstv, hist, tmp, accv, red):
        c = lax.axis_index("c")
        s = lax.axis_index("s")
        wid = c * NS + s

        pltpu.sync_copy(dst_hbm.at[pl.ds(wid * epw, epw)], dstv)

        @pl.loop(0, N, step=16)
        def _(i):
            hist[pl.ds(i, 16)] = jnp.zeros((16,), jnp.float32)

        ones = jnp.ones((16,), jnp.float32)

        @pl.loop(0, epw, step=16)
        def _(i):
            plsc.addupdate_scatter(hist, [dstv[pl.ds(i, 16)]], ones)

        pltpu.sync_copy(hist, red.at[s])
        plsc.subcore_barrier()

        @pl.loop(0, RPS, step=16)
        def _(i):
            accv[pl.ds(i, 16)] = jnp.zeros((16,), jnp.float32)

        @pl.loop(0, NS)
        def _(r):
            pltpu.sync_copy(red.at[r, pl.ds(s * RPS, RPS)], tmp)

            @pl.loop(0, RPS, step=16)
            def _(i):
                accv[pl.ds(i, 16)] = accv[pl.ds(i, 16)] + tmp[pl.ds(i, 16)]

        pltpu.sync_copy(accv, cnt_hbm.at[pl.ds(c * N + s * RPS, RPS)])

    return k(dstf)


def _tc_layer(agg3, h3, cnt2, wlt, wrt, bl2, gg2, bb2):
    R = 512

    def body(agg_ref, h_ref, cnt_ref, wl_ref, wr_ref, b_ref, g_ref, be_ref,
             o_ref):
        agg = jnp.concatenate([agg_ref[0], agg_ref[1]], axis=1)
        h = jnp.concatenate([h_ref[0], h_ref[1]], axis=1)
        cnt = cnt_ref[0] + cnt_ref[1]
        recip = 1.0 / jnp.maximum(cnt, 1.0)
        z = (jnp.dot(agg * recip, wl_ref[...],
                     preferred_element_type=jnp.float32,
                     precision=lax.Precision.HIGHEST)
             + jnp.dot(h, wr_ref[...],
                       preferred_element_type=jnp.float32,
                       precision=lax.Precision.HIGHEST)
             + b_ref[...])
        mu = jnp.mean(z, axis=1, keepdims=True)
        zc = z - mu
        var = jnp.mean(zc * zc, axis=1, keepdims=True)
        hn = zc / jnp.sqrt(var + 1e-5) * g_ref[...] + be_ref[...]
        hn = jnp.maximum(hn, 0.0)
        o_ref[0] = hn[:, :F]
        o_ref[1] = hn[:, F:]

    return pl.pallas_call(
        body,
        grid=(N // R,),
        in_specs=[
            pl.BlockSpec((NC, R, F), lambda i: (0, i, 0)),
            pl.BlockSpec((NC, R, F), lambda i: (0, i, 0)),
            pl.BlockSpec((NC, R, 1), lambda i: (0, i, 0)),
            pl.BlockSpec((H, H), lambda i: (0, 0)),
            pl.BlockSpec((H, H), lambda i: (0, 0)),
            pl.BlockSpec((1, H), lambda i: (0, 0)),
            pl.BlockSpec((1, H), lambda i: (0, 0)),
            pl.BlockSpec((1, H), lambda i: (0, 0)),
        ],
        out_specs=pl.BlockSpec((NC, R, F), lambda i: (0, i, 0)),
        out_shape=jax.ShapeDtypeStruct((NC, N, F), jnp.float32),
    )(agg3, h3, cnt2, wlt, wrt, bl2, gg2, bb2)


def _tc_pool(h3, batch2, gfeat, wm1t, bm12, wm2t, bm22):
    K = 256
    nblk = N // K

    def body(h_ref, b_ref, gf_ref, w1_ref, b1_ref, w2_ref, b2_ref, o_ref,
             sum_s, max_s, cnt_s):
        i = pl.program_id(0)

        @pl.when(i == 0)
        def _():
            sum_s[...] = jnp.zeros_like(sum_s)
            max_s[...] = jnp.full_like(max_s, -3.4e38)
            cnt_s[...] = jnp.zeros_like(cnt_s)

        h = jnp.concatenate([h_ref[0], h_ref[1]], axis=1)
        b = b_ref[...]
        glo = jnp.min(b)
        ghi = jnp.max(b)

        def gbody(g, carry):
            mrow = b == g
            csum = jnp.sum(jnp.where(mrow, h, 0.0), axis=0, keepdims=True)
            cmax = jnp.max(jnp.where(mrow, h, -3.4e38), axis=0, keepdims=True)
            ccnt = jnp.sum(mrow.astype(jnp.float32))
            sum_s[pl.ds(g, 1), :] += csum
            max_s[pl.ds(g, 1), :] = jnp.maximum(max_s[pl.ds(g, 1), :], cmax)
            cnt_s[pl.ds(g, 1), :] += ccnt
            return carry

        lax.fori_loop(glo, ghi + 1, gbody, 0)

        @pl.when(i == nblk - 1)
        def _():
            cnt = cnt_s[:, 0:1]
            mean = sum_s[...] / jnp.maximum(cnt, 1.0)
            mx = jnp.where(cnt > 0.0, max_s[...], 0.0)
            gcat = jnp.concatenate([mean, mx, gf_ref[...]], axis=1)
            hm = jnp.maximum(
                jnp.dot(gcat, w1_ref[...],
                        preferred_element_type=jnp.float32,
                        precision=lax.Precision.HIGHEST) + b1_ref[...], 0.0)
            o_ref[...] = jnp.dot(hm, w2_ref[...],
                                 preferred_element_type=jnp.float32,
                                 precision=lax.Precision.HIGHEST) + b2_ref[...]

    return pl.pallas_call(
        body,
        grid=(nblk,),
        in_specs=[
            pl.BlockSpec((NC, K, F), lambda i: (0, i, 0)),
            pl.BlockSpec((K, 1), lambda i: (i, 0)),
            pl.BlockSpec((G, GF), lambda i: (0, 0)),
            pl.BlockSpec((2 * H + GF, H), lambda i: (0, 0)),
            pl.BlockSpec((1, H), lambda i: (0, 0)),
            pl.BlockSpec((H, C), lambda i: (0, 0)),
            pl.BlockSpec((1, C), lambda i: (0, 0)),
        ],
        out_specs=pl.BlockSpec((G, C), lambda i: (0, 0)),
        out_shape=jax.ShapeDtypeStruct((G, C), jnp.float32),
        scratch_shapes=[
            pltpu.VMEM((G, H), jnp.float32),
            pltpu.VMEM((G, H), jnp.float32),
            pltpu.VMEM((G, H), jnp.float32),
        ],
    )(h3, batch2, gfeat, wm1t, bm12, wm2t, bm22)


def kernel(x, edge_index, batch, ptr, root_idx, gfeat, W_l0, b_l0, W_r0, g0,
           be0, W_l1, b_l1, W_r1, g1, be1, W_l2, b_l2, W_r2, g2, be2, Wm1,
           bm1, Wm2, bm2):
    src = edge_index[0]
    dst = edge_index[1]
    src_r = src.reshape(NS * NCHG, CH)
    srcm = jnp.concatenate([src_r, src_r + N], axis=0)
    dstm = dst.reshape(NS * NCHG, CH)

    zeros = jnp.zeros((N, F), jnp.float32)

    cntf = _sc_count(dst)
    cnt3 = cntf.reshape(NC, N, 1)

    h3 = x.reshape(N, NC, F).transpose(1, 0, 2)
    params = [(W_l0, b_l0, W_r0, g0, be0),
              (W_l1, b_l1, W_r1, g1, be1),
              (W_l2, b_l2, W_r2, g2, be2)]
    for wl, bl, wr, gg, bb in params:
        aggf = _sc_agg(h3.reshape(NC * N, F), srcm, dstm, zeros)
        h3 = _tc_layer(aggf.reshape(NC, N, F), h3, cnt3, wl.T, wr.T,
                       bl.reshape(1, H), gg.reshape(1, H), bb.reshape(1, H))

    return _tc_pool(h3, batch.reshape(N, 1), gfeat, Wm1.T,
                    bm1.reshape(1, H), Wm2.T, bm2.reshape(1, C))


# speedup vs baseline: 1.0667x; 1.0074x over previous
import dataclasses
import functools

import jax
import jax.numpy as jnp
from jax import lax
from jax.experimental import pallas as pl
from jax.experimental.pallas import tpu as pltpu
from jax.experimental.pallas import tpu_sc as plsc

N = 10240
E = 163840
D = 256
H = 256
G = 64
C = 10
GF = 16

F = 128
NC = 2
NS = 16
CH = 128
EPW = E // NS
NCHG = EPW // CH
RPS = N // NS


def _sc_agg(h2, srcm, dstm, zeros):
    mesh = plsc.VectorSubcoreMesh(core_axis_name="c", subcore_axis_name="s")

    @functools.partial(
        pl.kernel,
        out_type=jax.ShapeDtypeStruct((NC * N, F), jnp.float32),
        mesh=mesh,
        scratch_types=[
            pltpu.VMEM((4, CH), jnp.int32),
            pltpu.VMEM((4, CH), jnp.int32),
            pltpu.VMEM((4, CH), jnp.int32),
            pltpu.VMEM((4, CH), jnp.int32),
            pltpu.VMEM((CH, F), jnp.float32),
            pltpu.VMEM((CH, F), jnp.float32),
            pltpu.SemaphoreType.DMA,
            pltpu.SemaphoreType.DMA,
            pltpu.SemaphoreType.DMA,
            pltpu.SemaphoreType.DMA,
            pltpu.SemaphoreType.DMA,
            pltpu.SemaphoreType.DMA,
            pltpu.VMEM_SHARED((N, F), jnp.float32),
        ],
    )
    def k(h2_hbm, srcm_hbm, dstm_hbm, zeros_hbm, out_hbm,
          svA, dvA, svB, dvB, rows0, rows1,
          sem0, sem1, sem_s0, sem_s1, sem_iA, sem_iB, acc):
        c = lax.axis_index("c")
        s = lax.axis_index("s")
        GS = 4
        NG = NCHG // GS

        pltpu.sync_copy(zeros_hbm.at[pl.ds(s * RPS, RPS)],
                        acc.at[pl.ds(s * RPS, RPS)])
        plsc.subcore_barrier()

        gbase = (c * NS + s) * NCHG
        dbase = s * NCHG

        def stage_start(g, sv, dv, sem):
            pltpu.async_copy(srcm_hbm.at[pl.ds(gbase + g * GS, GS)], sv, sem)
            pltpu.async_copy(dstm_hbm.at[pl.ds(dbase + g * GS, GS)], dv, sem)

        def stage_wait(g, sv, dv, sem):
            pltpu.make_async_copy(srcm_hbm.at[pl.ds(gbase + g * GS, GS)],
                                  sv, sem).wait()
            pltpu.make_async_copy(dstm_hbm.at[pl.ds(dbase + g * GS, GS)],
                                  dv, sem).wait()

        def g_start(sv, k, rows, sem):
            pltpu.async_copy(h2_hbm.at[sv.at[k]], rows, sem)

        def g_wait(sv, k, rows, sem):
            pltpu.make_async_copy(h2_hbm.at[sv.at[k]], rows, sem).wait()

        def s_start(rows, dv, k, sem):
            return pltpu.async_copy(rows, acc.at[dv.at[k]], sem, add=True)

        stage_start(0, svA, dvA, sem_iA)
        stage_wait(0, svA, dvA, sem_iA)
        g_start(svA, 0, rows0, sem0)

        @pl.loop(0, NG, step=2)
        def _(g):
            scat = [None, None]
            for k in range(8):
                p, q = k % 2, 1 - k % 2
                sv, dv, ksub = (svA, dvA, k) if k < 4 else (svB, dvB, k - 4)
                rows, gsem = (rows0, sem0) if p == 0 else (rows1, sem1)
                ssem = sem_s0 if p == 0 else sem_s1
                g_wait(sv, ksub, rows, gsem)
                scat[p] = s_start(rows, dv, ksub, ssem)
                if k == 0:
                    stage_start(g + 1, svB, dvB, sem_iB)
                if scat[q] is not None:
                    scat[q].wait()
                    scat[q] = None
                if k == 3:
                    stage_wait(g + 1, svB, dvB, sem_iB)
                if k == 5:
                    @pl.when(g + 2 < NG)
                    def _():
                        stage_start(g + 2, svA, dvA, sem_iA)
                if k < 7:
                    nsv, nk = (svA, k + 1) if k < 3 else (svB, k - 3)
                    nrows, nsem = (rows1, sem1) if p == 0 else (rows0, sem0)
                    g_start(nsv, nk, nrows, nsem)
                else:
                    @pl.when(g + 2 < NG)
                    def _():
                        stage_wait(g + 2, svA, dvA, sem_iA)
                        g_start(svA, 0, rows0, sem0)
            scat[1].wait()

        plsc.subcore_barrier()
        pltpu.sync_copy(acc.at[pl.ds(s * RPS, RPS)],
                        out_hbm.at[pl.ds(c * N + s * RPS, RPS)])

    return k(h2, srcm, dstm, zeros)


def _sc_count(dstf):
    mesh = plsc.VectorSubcoreMesh(core_axis_name="c", subcore_axis_name="s")
    epw = E // (NC * NS)

    @functools.partial(
        pl.kernel,
        out_type=jax.ShapeDtypeStruct((NC * N,), jnp.float32),
        mesh=mesh,
        scratch_types=[
            pltpu.VMEM((epw,), jnp.int32),
            pltpu.VMEM((N,), jnp.float32),
            pltpu.VMEM((RPS,), jnp.float32),
            pltpu.VMEM((RPS,), jnp.float32),
            pltpu.VMEM_SHARED((NS, N), jnp.float32),
        ],
        compiler_params=dataclasses.replace(pltpu.CompilerParams(),
                                            needs_layout_passes=False),
    )
    def k(dst_hbm, cnt_hbm, dstv, hist, tmp, accv, red):
        c = lax.axis_index("c")
        s = lax.axis_index("s")
        wid = c * NS + s

        pltpu.sync_copy(dst_hbm.at[pl.ds(wid * epw, epw)], dstv)

        @pl.loop(0, N, step=16)
        def _(i):
            hist[pl.ds(i, 16)] = jnp.zeros((16,), jnp.float32)

        ones = jnp.ones((16,), jnp.float32)

        @pl.loop(0, epw, step=16)
        def _(i):
            plsc.addupdate_scatter(hist, [dstv[pl.ds(i, 16)]], ones)

        pltpu.sync_copy(hist, red.at[s])
        plsc.subcore_barrier()

        @pl.loop(0, RPS, step=16)
        def _(i):
            accv[pl.ds(i, 16)] = jnp.zeros((16,), jnp.float32)

        @pl.loop(0, NS)
        def _(r):
            pltpu.sync_copy(red.at[r, pl.ds(s * RPS, RPS)], tmp)

            @pl.loop(0, RPS, step=16)
            def _(i):
                accv[pl.ds(i, 16)] = accv[pl.ds(i, 16)] + tmp[pl.ds(i, 16)]

        pltpu.sync_copy(accv, cnt_hbm.at[pl.ds(c * N + s * RPS, RPS)])

    return k(dstf)


def _tc_layer(agg3, h3, cnt2, wlt, wrt, bl2, gg2, bb2):
    R = 512

    def body(agg_ref, h_ref, cnt_ref, wl_ref, wr_ref, b_ref, g_ref, be_ref,
             o_ref):
        agg = jnp.concatenate([agg_ref[0], agg_ref[1]], axis=1)
        h = jnp.concatenate([h_ref[0], h_ref[1]], axis=1)
        cnt = cnt_ref[0] + cnt_ref[1]
        recip = 1.0 / jnp.maximum(cnt, 1.0)
        z = (jnp.dot(agg * recip, wl_ref[...],
                     preferred_element_type=jnp.float32,
                     precision=lax.Precision.HIGHEST)
             + jnp.dot(h, wr_ref[...],
                       preferred_element_type=jnp.float32,
                       precision=lax.Precision.HIGHEST)
             + b_ref[...])
        mu = jnp.mean(z, axis=1, keepdims=True)
        zc = z - mu
        var = jnp.mean(zc * zc, axis=1, keepdims=True)
        hn = zc / jnp.sqrt(var + 1e-5) * g_ref[...] + be_ref[...]
        hn = jnp.maximum(hn, 0.0)
        o_ref[0] = hn[:, :F]
        o_ref[1] = hn[:, F:]

    return pl.pallas_call(
        body,
        grid=(N // R,),
        in_specs=[
            pl.BlockSpec((NC, R, F), lambda i: (0, i, 0)),
            pl.BlockSpec((NC, R, F), lambda i: (0, i, 0)),
            pl.BlockSpec((NC, R, 1), lambda i: (0, i, 0)),
            pl.BlockSpec((H, H), lambda i: (0, 0)),
            pl.BlockSpec((H, H), lambda i: (0, 0)),
            pl.BlockSpec((1, H), lambda i: (0, 0)),
            pl.BlockSpec((1, H), lambda i: (0, 0)),
            pl.BlockSpec((1, H), lambda i: (0, 0)),
        ],
        out_specs=pl.BlockSpec((NC, R, F), lambda i: (0, i, 0)),
        out_shape=jax.ShapeDtypeStruct((NC, N, F), jnp.float32),
    )(agg3, h3, cnt2, wlt, wrt, bl2, gg2, bb2)


def _tc_pool(h3, batch2, gfeat, wm1t, bm12, wm2t, bm22):
    K = 256
    nblk = N // K

    def body(h_ref, b_ref, gf_ref, w1_ref, b1_ref, w2_ref, b2_ref, o_ref,
             sum_s, max_s, cnt_s):
        i = pl.program_id(0)

        @pl.when(i == 0)
        def _():
            sum_s[...] = jnp.zeros_like(sum_s)
            max_s[...] = jnp.full_like(max_s, -3.4e38)
            cnt_s[...] = jnp.zeros_like(cnt_s)

        h = jnp.concatenate([h_ref[0], h_ref[1]], axis=1)
        b = b_ref[...]
        glo = jnp.min(b)
        ghi = jnp.max(b)

        def gbody(g, carry):
            mrow = b == g
            csum = jnp.sum(jnp.where(mrow, h, 0.0), axis=0, keepdims=True)
            cmax = jnp.max(jnp.where(mrow, h, -3.4e38), axis=0, keepdims=True)
            ccnt = jnp.sum(mrow.astype(jnp.float32))
            sum_s[pl.ds(g, 1), :] += csum
            max_s[pl.ds(g, 1), :] = jnp.maximum(max_s[pl.ds(g, 1), :], cmax)
            cnt_s[pl.ds(g, 1), :] += ccnt
            return carry

        lax.fori_loop(glo, ghi + 1, gbody, 0)

        @pl.when(i == nblk - 1)
        def _():
            cnt = cnt_s[:, 0:1]
            mean = sum_s[...] / jnp.maximum(cnt, 1.0)
            mx = jnp.where(cnt > 0.0, max_s[...], 0.0)
            gcat = jnp.concatenate([mean, mx, gf_ref[...]], axis=1)
            hm = jnp.maximum(
                jnp.dot(gcat, w1_ref[...],
                        preferred_element_type=jnp.float32,
                        precision=lax.Precision.HIGHEST) + b1_ref[...], 0.0)
            o_ref[...] = jnp.dot(hm, w2_ref[...],
                                 preferred_element_type=jnp.float32,
                                 precision=lax.Precision.HIGHEST) + b2_ref[...]

    return pl.pallas_call(
        body,
        grid=(nblk,),
        in_specs=[
            pl.BlockSpec((NC, K, F), lambda i: (0, i, 0)),
            pl.BlockSpec((K, 1), lambda i: (i, 0)),
            pl.BlockSpec((G, GF), lambda i: (0, 0)),
            pl.BlockSpec((2 * H + GF, H), lambda i: (0, 0)),
            pl.BlockSpec((1, H), lambda i: (0, 0)),
            pl.BlockSpec((H, C), lambda i: (0, 0)),
            pl.BlockSpec((1, C), lambda i: (0, 0)),
        ],
        out_specs=pl.BlockSpec((G, C), lambda i: (0, 0)),
        out_shape=jax.ShapeDtypeStruct((G, C), jnp.float32),
        scratch_shapes=[
            pltpu.VMEM((G, H), jnp.float32),
            pltpu.VMEM((G, H), jnp.float32),
            pltpu.VMEM((G, H), jnp.float32),
        ],
    )(h3, batch2, gfeat, wm1t, bm12, wm2t, bm22)


def kernel(x, edge_index, batch, ptr, root_idx, gfeat, W_l0, b_l0, W_r0, g0,
           be0, W_l1, b_l1, W_r1, g1, be1, W_l2, b_l2, W_r2, g2, be2, Wm1,
           bm1, Wm2, bm2):
    src = edge_index[0]
    dst = edge_index[1]
    src_r = src.reshape(NS * NCHG, CH)
    srcm = jnp.concatenate([src_r, src_r + N], axis=0)
    dstm = dst.reshape(NS * NCHG, CH)

    zeros = jnp.zeros((N, F), jnp.float32)

    cntf = _sc_count(dst)
    cnt3 = cntf.reshape(NC, N, 1)

    h3 = x.reshape(N, NC, F).transpose(1, 0, 2)
    params = [(W_l0, b_l0, W_r0, g0, be0),
              (W_l1, b_l1, W_r1, g1, be1),
              (W_l2, b_l2, W_r2, g2, be2)]
    for wl, bl, wr, gg, bb in params:
        aggf = _sc_agg(h3.reshape(NC * N, F), srcm, dstm, zeros)
        h3 = _tc_layer(aggf.reshape(NC, N, F), h3, cnt3, wl.T, wr.T,
                       bl.reshape(1, H), gg.reshape(1, H), bb.reshape(1, H))

    return _tc_pool(h3, batch.reshape(N, 1), gfeat, Wm1.T,
                    bm1.reshape(1, H), Wm2.T, bm2.reshape(1, C))


# speedup vs baseline: 1.0978x; 1.0291x over previous
import dataclasses
import functools

import jax
import jax.numpy as jnp
from jax import lax
from jax.experimental import pallas as pl
from jax.experimental.pallas import tpu as pltpu
from jax.experimental.pallas import tpu_sc as plsc

N = 10240
E = 163840
D = 256
H = 256
G = 64
C = 10
GF = 16

F = 128
NC = 2
NS = 16
CH = 128
EPW = E // NS
NCHG = EPW // CH
RPS = N // NS


def _sc_agg(h2, srcm, dstm, zeros):
    mesh = plsc.VectorSubcoreMesh(core_axis_name="c", subcore_axis_name="s")

    @functools.partial(
        pl.kernel,
        out_type=jax.ShapeDtypeStruct((NC * N, F), jnp.float32),
        mesh=mesh,
        scratch_types=[
            pltpu.VMEM((4, CH), jnp.int32),
            pltpu.VMEM((4, CH), jnp.int32),
            pltpu.VMEM((4, CH), jnp.int32),
            pltpu.VMEM((4, CH), jnp.int32),
            pltpu.VMEM((CH, F), jnp.float32),
            pltpu.VMEM((CH, F), jnp.float32),
            pltpu.SemaphoreType.DMA,
            pltpu.SemaphoreType.DMA,
            pltpu.SemaphoreType.DMA,
            pltpu.SemaphoreType.DMA,
            pltpu.SemaphoreType.DMA,
            pltpu.SemaphoreType.DMA,
            pltpu.VMEM_SHARED((N, F), jnp.float32),
        ],
    )
    def k(h2_hbm, srcm_hbm, dstm_hbm, zeros_hbm, out_hbm,
          svA, dvA, svB, dvB, rows0, rows1,
          sem0, sem1, sem_s0, sem_s1, sem_iA, sem_iB, acc):
        c = lax.axis_index("c")
        s = lax.axis_index("s")
        GS = 4
        NG = NCHG // GS

        pltpu.sync_copy(zeros_hbm.at[pl.ds(s * RPS, RPS)],
                        acc.at[pl.ds(s * RPS, RPS)])
        plsc.subcore_barrier()

        gbase = (c * NS + s) * NCHG
        dbase = s * NCHG

        def stage_start(g, sv, dv, sem):
            pltpu.async_copy(srcm_hbm.at[pl.ds(gbase + g * GS, GS)], sv, sem)
            pltpu.async_copy(dstm_hbm.at[pl.ds(dbase + g * GS, GS)], dv, sem)

        def stage_wait(g, sv, dv, sem):
            pltpu.make_async_copy(srcm_hbm.at[pl.ds(gbase + g * GS, GS)],
                                  sv, sem).wait()
            pltpu.make_async_copy(dstm_hbm.at[pl.ds(dbase + g * GS, GS)],
                                  dv, sem).wait()

        def g_start(sv, k, rows, sem):
            pltpu.async_copy(h2_hbm.at[sv.at[k]], rows, sem)

        def g_wait(sv, k, rows, sem):
            pltpu.make_async_copy(h2_hbm.at[sv.at[k]], rows, sem).wait()

        def s_start(rows, dv, k, sem):
            return pltpu.async_copy(rows, acc.at[dv.at[k]], sem, add=True)

        stage_start(0, svA, dvA, sem_iA)
        stage_wait(0, svA, dvA, sem_iA)
        g_start(svA, 0, rows0, sem0)

        @pl.loop(0, NG, step=2)
        def _(g):
            scat = [None, None]
            for k in range(8):
                p, q = k % 2, 1 - k % 2
                sv, dv, ksub = (svA, dvA, k) if k < 4 else (svB, dvB, k - 4)
                rows, gsem = (rows0, sem0) if p == 0 else (rows1, sem1)
                ssem = sem_s0 if p == 0 else sem_s1
                g_wait(sv, ksub, rows, gsem)
                scat[p] = s_start(rows, dv, ksub, ssem)
                if k == 0:
                    stage_start(g + 1, svB, dvB, sem_iB)
                if scat[q] is not None:
                    scat[q].wait()
                    scat[q] = None
                if k == 3:
                    stage_wait(g + 1, svB, dvB, sem_iB)
                if k == 5:
                    @pl.when(g + 2 < NG)
                    def _():
                        stage_start(g + 2, svA, dvA, sem_iA)
                if k < 7:
                    nsv, nk = (svA, k + 1) if k < 3 else (svB, k - 3)
                    nrows, nsem = (rows1, sem1) if p == 0 else (rows0, sem0)
                    g_start(nsv, nk, nrows, nsem)
                else:
                    @pl.when(g + 2 < NG)
                    def _():
                        stage_wait(g + 2, svA, dvA, sem_iA)
                        g_start(svA, 0, rows0, sem0)
            scat[1].wait()

        plsc.subcore_barrier()
        pltpu.sync_copy(acc.at[pl.ds(s * RPS, RPS)],
                        out_hbm.at[pl.ds(c * N + s * RPS, RPS)])

    return k(h2, srcm, dstm, zeros)


def _sc_count(dstf):
    mesh = plsc.VectorSubcoreMesh(core_axis_name="c", subcore_axis_name="s")
    epw = E // (NC * NS)

    @functools.partial(
        pl.kernel,
        out_type=jax.ShapeDtypeStruct((NC * N,), jnp.float32),
        mesh=mesh,
        scratch_types=[
            pltpu.VMEM((epw,), jnp.int32),
            pltpu.VMEM((N,), jnp.float32),
            pltpu.VMEM((RPS,), jnp.float32),
            pltpu.VMEM((RPS,), jnp.float32),
            pltpu.VMEM_SHARED((NS, N), jnp.float32),
        ],
        compiler_params=dataclasses.replace(pltpu.CompilerParams(),
                                            needs_layout_passes=False),
    )
    def k(dst_hbm, cnt_hbm, dstv, hist, tmp, accv, red):
        c = lax.axis_index("c")
        s = lax.axis_index("s")
        wid = c * NS + s

        pltpu.sync_copy(dst_hbm.at[pl.ds(wid * epw, epw)], dstv)

        @pl.loop(0, N, step=16)
        def _(i):
            hist[pl.ds(i, 16)] = jnp.zeros((16,), jnp.float32)

        ones = jnp.ones((16,), jnp.float32)

        @pl.loop(0, epw, step=16)
        def _(i):
            plsc.addupdate_scatter(hist, [dstv[pl.ds(i, 16)]], ones)

        pltpu.sync_copy(hist, red.at[s])
        plsc.subcore_barrier()

        @pl.loop(0, RPS, step=16)
        def _(i):
            accv[pl.ds(i, 16)] = jnp.zeros((16,), jnp.float32)

        @pl.loop(0, NS)
        def _(r):
            pltpu.sync_copy(red.at[r, pl.ds(s * RPS, RPS)], tmp)

            @pl.loop(0, RPS, step=16)
            def _(i):
                accv[pl.ds(i, 16)] = accv[pl.ds(i, 16)] + tmp[pl.ds(i, 16)]

        pltpu.sync_copy(accv, cnt_hbm.at[pl.ds(c * N + s * RPS, RPS)])

    return k(dstf)


def _tc_layer(agg3, h3, cnt2, wlt, wrt, bl2, gg2, bb2):
    R = 512

    def body(agg_ref, h_ref, cnt_ref, wl_ref, wr_ref, b_ref, g_ref, be_ref,
             o_ref):
        agg = jnp.concatenate([agg_ref[0], agg_ref[1]], axis=1)
        h = jnp.concatenate([h_ref[0], h_ref[1]], axis=1)
        cnt = cnt_ref[0] + cnt_ref[1]
        recip = 1.0 / jnp.maximum(cnt, 1.0)
        z = (jnp.dot(agg * recip, wl_ref[...],
                     preferred_element_type=jnp.float32,
                     precision=lax.Precision.HIGHEST)
             + jnp.dot(h, wr_ref[...],
                       preferred_element_type=jnp.float32,
                       precision=lax.Precision.HIGHEST)
             + b_ref[...])
        mu = jnp.mean(z, axis=1, keepdims=True)
        zc = z - mu
        var = jnp.mean(zc * zc, axis=1, keepdims=True)
        hn = zc / jnp.sqrt(var + 1e-5) * g_ref[...] + be_ref[...]
        hn = jnp.maximum(hn, 0.0)
        o_ref[0] = hn[:, :F]
        o_ref[1] = hn[:, F:]

    return pl.pallas_call(
        body,
        grid=(N // R,),
        in_specs=[
            pl.BlockSpec((NC, R, F), lambda i: (0, i, 0)),
            pl.BlockSpec((NC, R, F), lambda i: (0, i, 0)),
            pl.BlockSpec((NC, R, 1), lambda i: (0, i, 0)),
            pl.BlockSpec((H, H), lambda i: (0, 0)),
            pl.BlockSpec((H, H), lambda i: (0, 0)),
            pl.BlockSpec((1, H), lambda i: (0, 0)),
            pl.BlockSpec((1, H), lambda i: (0, 0)),
            pl.BlockSpec((1, H), lambda i: (0, 0)),
        ],
        out_specs=pl.BlockSpec((NC, R, F), lambda i: (0, i, 0)),
        out_shape=jax.ShapeDtypeStruct((NC, N, F), jnp.float32),
    )(agg3, h3, cnt2, wlt, wrt, bl2, gg2, bb2)


def _tc_layer3_pool(agg3, h3, cnt3, wlt, wrt, bl2, gg2, bb2, batch2, gfeat,
                    wm1t, bm12, wm2t, bm22):
    R = 512
    nblk = N // R

    def body(agg_ref, h_ref, cnt_ref, wl_ref, wr_ref, b_ref, g_ref, be_ref,
             bt_ref, gf_ref, w1_ref, b1_ref, w2_ref, b2_ref, o_ref,
             sum_s, max_s, cnt_s):
        i = pl.program_id(0)

        @pl.when(i == 0)
        def _():
            sum_s[...] = jnp.zeros_like(sum_s)
            max_s[...] = jnp.full_like(max_s, -3.4e38)
            cnt_s[...] = jnp.zeros_like(cnt_s)

        agg = jnp.concatenate([agg_ref[0], agg_ref[1]], axis=1)
        h = jnp.concatenate([h_ref[0], h_ref[1]], axis=1)
        cnt = cnt_ref[0] + cnt_ref[1]
        recip = 1.0 / jnp.maximum(cnt, 1.0)
        z = (jnp.dot(agg * recip, wl_ref[...],
                     preferred_element_type=jnp.float32,
                     precision=lax.Precision.HIGHEST)
             + jnp.dot(h, wr_ref[...],
                       preferred_element_type=jnp.float32,
                       precision=lax.Precision.HIGHEST)
             + b_ref[...])
        mu = jnp.mean(z, axis=1, keepdims=True)
        zc = z - mu
        var = jnp.mean(zc * zc, axis=1, keepdims=True)
        hn = zc / jnp.sqrt(var + 1e-5) * g_ref[...] + be_ref[...]
        hn = jnp.maximum(hn, 0.0)

        b = bt_ref[...]
        glo = jnp.min(b)
        ghi = jnp.max(b)

        def gbody(g, carry):
            mrow = b == g
            csum = jnp.sum(jnp.where(mrow, hn, 0.0), axis=0, keepdims=True)
            cmax = jnp.max(jnp.where(mrow, hn, -3.4e38), axis=0,
                           keepdims=True)
            ccnt = jnp.sum(mrow.astype(jnp.float32))
            sum_s[pl.ds(g, 1), :] += csum
            max_s[pl.ds(g, 1), :] = jnp.maximum(max_s[pl.ds(g, 1), :], cmax)
            cnt_s[pl.ds(g, 1), :] += ccnt
            return carry

        lax.fori_loop(glo, ghi + 1, gbody, 0)

        @pl.when(i == nblk - 1)
        def _():
            pcnt = cnt_s[:, 0:1]
            mean = sum_s[...] / jnp.maximum(pcnt, 1.0)
            mx = jnp.where(pcnt > 0.0, max_s[...], 0.0)
            gcat = jnp.concatenate([mean, mx, gf_ref[...]], axis=1)
            hm = jnp.maximum(
                jnp.dot(gcat, w1_ref[...],
                        preferred_element_type=jnp.float32,
                        precision=lax.Precision.HIGHEST) + b1_ref[...], 0.0)
            o_ref[...] = jnp.dot(hm, w2_ref[...],
                                 preferred_element_type=jnp.float32,
                                 precision=lax.Precision.HIGHEST) + b2_ref[...]

    return pl.pallas_call(
        body,
        grid=(nblk,),
        in_specs=[
            pl.BlockSpec((NC, R, F), lambda i: (0, i, 0)),
            pl.BlockSpec((NC, R, F), lambda i: (0, i, 0)),
            pl.BlockSpec((NC, R, 1), lambda i: (0, i, 0)),
            pl.BlockSpec((H, H), lambda i: (0, 0)),
            pl.BlockSpec((H, H), lambda i: (0, 0)),
            pl.BlockSpec((1, H), lambda i: (0, 0)),
            pl.BlockSpec((1, H), lambda i: (0, 0)),
            pl.BlockSpec((1, H), lambda i: (0, 0)),
            pl.BlockSpec((R, 1), lambda i: (i, 0)),
            pl.BlockSpec((G, GF), lambda i: (0, 0)),
            pl.BlockSpec((2 * H + GF, H), lambda i: (0, 0)),
            pl.BlockSpec((1, H), lambda i: (0, 0)),
            pl.BlockSpec((H, C), lambda i: (0, 0)),
            pl.BlockSpec((1, C), lambda i: (0, 0)),
        ],
        out_specs=pl.BlockSpec((G, C), lambda i: (0, 0)),
        out_shape=jax.ShapeDtypeStruct((G, C), jnp.float32),
        scratch_shapes=[
            pltpu.VMEM((G, H), jnp.float32),
            pltpu.VMEM((G, H), jnp.float32),
            pltpu.VMEM((G, H), jnp.float32),
        ],
    )(agg3, h3, cnt3, wlt, wrt, bl2, gg2, bb2, batch2, gfeat, wm1t, bm12,
      wm2t, bm22)


def kernel(x, edge_index, batch, ptr, root_idx, gfeat, W_l0, b_l0, W_r0, g0,
           be0, W_l1, b_l1, W_r1, g1, be1, W_l2, b_l2, W_r2, g2, be2, Wm1,
           bm1, Wm2, bm2):
    src = edge_index[0]
    dst = edge_index[1]
    src_r = src.reshape(NS * NCHG, CH)
    srcm = jnp.concatenate([src_r, src_r + N], axis=0)
    dstm = dst.reshape(NS * NCHG, CH)

    zeros = jnp.zeros((N, F), jnp.float32)

    cntf = _sc_count(dst)
    cnt3 = cntf.reshape(NC, N, 1)

    h3 = x.reshape(N, NC, F).transpose(1, 0, 2)
    for wl, bl, wr, gg, bb in [(W_l0, b_l0, W_r0, g0, be0),
                               (W_l1, b_l1, W_r1, g1, be1)]:
        aggf = _sc_agg(h3.reshape(NC * N, F), srcm, dstm, zeros)
        h3 = _tc_layer(aggf.reshape(NC, N, F), h3, cnt3, wl.T, wr.T,
                       bl.reshape(1, H), gg.reshape(1, H), bb.reshape(1, H))

    aggf = _sc_agg(h3.reshape(NC * N, F), srcm, dstm, zeros)
    return _tc_layer3_pool(aggf.reshape(NC, N, F), h3, cnt3, W_l2.T, W_r2.T,
                           b_l2.reshape(1, H), g2.reshape(1, H),
                           be2.reshape(1, H), batch.reshape(N, 1), gfeat,
                           Wm1.T, bm1.reshape(1, H), Wm2.T,
                           bm2.reshape(1, C))


# speedup vs baseline: 1.1262x; 1.0259x over previous
import dataclasses
import functools

import jax
import jax.numpy as jnp
from jax import lax
from jax.experimental import pallas as pl
from jax.experimental.pallas import tpu as pltpu
from jax.experimental.pallas import tpu_sc as plsc

N = 10240
E = 163840
D = 256
H = 256
G = 64
C = 10
GF = 16

F = 128
NC = 2
NS = 16
CH = 128
EPW = E // NS
NCHG = EPW // CH
RPS = N // NS


def _sc_agg(h2, srcm, dstm, zeros):
    mesh = plsc.VectorSubcoreMesh(core_axis_name="c", subcore_axis_name="s")

    @functools.partial(
        pl.kernel,
        out_type=jax.ShapeDtypeStruct((NC * N, F), jnp.float32),
        mesh=mesh,
        scratch_types=[
            pltpu.VMEM((4, CH), jnp.int32),
            pltpu.VMEM((4, CH), jnp.int32),
            pltpu.VMEM((4, CH), jnp.int32),
            pltpu.VMEM((4, CH), jnp.int32),
            pltpu.VMEM((CH, F), jnp.float32),
            pltpu.VMEM((CH, F), jnp.float32),
            pltpu.SemaphoreType.DMA,
            pltpu.SemaphoreType.DMA,
            pltpu.SemaphoreType.DMA,
            pltpu.SemaphoreType.DMA,
            pltpu.SemaphoreType.DMA,
            pltpu.SemaphoreType.DMA,
            pltpu.SemaphoreType.DMA,
            pltpu.VMEM_SHARED((N, F), jnp.float32),
        ],
    )
    def k(h2_hbm, srcm_hbm, dstm_hbm, zeros_hbm, out_hbm,
          svA, dvA, svB, dvB, rows0, rows1,
          sem0, sem1, sem_s0, sem_s1, sem_iA, sem_iB, sem_z, acc):
        c = lax.axis_index("c")
        s = lax.axis_index("s")
        GS = 4
        NG = NCHG // GS

        zd = pltpu.async_copy(zeros_hbm.at[pl.ds(s * RPS, RPS)],
                              acc.at[pl.ds(s * RPS, RPS)], sem_z)

        gbase = (c * NS + s) * NCHG
        dbase = s * NCHG

        def stage_start(g, sv, dv, sem):
            pltpu.async_copy(srcm_hbm.at[pl.ds(gbase + g * GS, GS)], sv, sem)
            pltpu.async_copy(dstm_hbm.at[pl.ds(dbase + g * GS, GS)], dv, sem)

        def stage_wait(g, sv, dv, sem):
            pltpu.make_async_copy(srcm_hbm.at[pl.ds(gbase + g * GS, GS)],
                                  sv, sem).wait()
            pltpu.make_async_copy(dstm_hbm.at[pl.ds(dbase + g * GS, GS)],
                                  dv, sem).wait()

        def g_start(sv, k, rows, sem):
            pltpu.async_copy(h2_hbm.at[sv.at[k]], rows, sem)

        def g_wait(sv, k, rows, sem):
            pltpu.make_async_copy(h2_hbm.at[sv.at[k]], rows, sem).wait()

        def s_start(rows, dv, k, sem):
            return pltpu.async_copy(rows, acc.at[dv.at[k]], sem, add=True)

        stage_start(0, svA, dvA, sem_iA)
        stage_wait(0, svA, dvA, sem_iA)
        g_start(svA, 0, rows0, sem0)
        zd.wait()
        plsc.subcore_barrier()

        @pl.loop(0, NG, step=2)
        def _(g):
            scat = [None, None]
            for k in range(8):
                p, q = k % 2, 1 - k % 2
                sv, dv, ksub = (svA, dvA, k) if k < 4 else (svB, dvB, k - 4)
                rows, gsem = (rows0, sem0) if p == 0 else (rows1, sem1)
                ssem = sem_s0 if p == 0 else sem_s1
                g_wait(sv, ksub, rows, gsem)
                scat[p] = s_start(rows, dv, ksub, ssem)
                if k == 0:
                    stage_start(g + 1, svB, dvB, sem_iB)
                if scat[q] is not None:
                    scat[q].wait()
                    scat[q] = None
                if k == 3:
                    stage_wait(g + 1, svB, dvB, sem_iB)
                if k == 5:
                    @pl.when(g + 2 < NG)
                    def _():
                        stage_start(g + 2, svA, dvA, sem_iA)
                if k < 7:
                    nsv, nk = (svA, k + 1) if k < 3 else (svB, k - 3)
                    nrows, nsem = (rows1, sem1) if p == 0 else (rows0, sem0)
                    g_start(nsv, nk, nrows, nsem)
                else:
                    @pl.when(g + 2 < NG)
                    def _():
                        stage_wait(g + 2, svA, dvA, sem_iA)
                        g_start(svA, 0, rows0, sem0)
            scat[1].wait()

        plsc.subcore_barrier()
        pltpu.sync_copy(acc.at[pl.ds(s * RPS, RPS)],
                        out_hbm.at[pl.ds(c * N + s * RPS, RPS)])

    return k(h2, srcm, dstm, zeros)


def _sc_count(dstf):
    mesh = plsc.VectorSubcoreMesh(core_axis_name="c", subcore_axis_name="s")
    epw = E // (NC * NS)

    @functools.partial(
        pl.kernel,
        out_type=jax.ShapeDtypeStruct((NC * N,), jnp.float32),
        mesh=mesh,
        scratch_types=[
            pltpu.VMEM((epw,), jnp.int32),
            pltpu.VMEM((N,), jnp.float32),
            pltpu.VMEM((RPS,), jnp.float32),
            pltpu.VMEM((RPS,), jnp.float32),
            pltpu.VMEM_SHARED((NS, N), jnp.float32),
        ],
        compiler_params=dataclasses.replace(pltpu.CompilerParams(),
                                            needs_layout_passes=False),
    )
    def k(dst_hbm, cnt_hbm, dstv, hist, tmp, accv, red):
        c = lax.axis_index("c")
        s = lax.axis_index("s")
        wid = c * NS + s

        pltpu.sync_copy(dst_hbm.at[pl.ds(wid * epw, epw)], dstv)

        @pl.loop(0, N, step=16)
        def _(i):
            hist[pl.ds(i, 16)] = jnp.zeros((16,), jnp.float32)

        ones = jnp.ones((16,), jnp.float32)

        @pl.loop(0, epw, step=16)
        def _(i):
            plsc.addupdate_scatter(hist, [dstv[pl.ds(i, 16)]], ones)

        pltpu.sync_copy(hist, red.at[s])
        plsc.subcore_barrier()

        @pl.loop(0, RPS, step=16)
        def _(i):
            accv[pl.ds(i, 16)] = jnp.zeros((16,), jnp.float32)

        @pl.loop(0, NS)
        def _(r):
            pltpu.sync_copy(red.at[r, pl.ds(s * RPS, RPS)], tmp)

            @pl.loop(0, RPS, step=16)
            def _(i):
                accv[pl.ds(i, 16)] = accv[pl.ds(i, 16)] + tmp[pl.ds(i, 16)]

        pltpu.sync_copy(accv, cnt_hbm.at[pl.ds(c * N + s * RPS, RPS)])

    return k(dstf)


def _tc_layer(agg3, h3, cnt2, wlt, wrt, bl2, gg2, bb2):
    R = 1024

    def body(agg_ref, h_ref, cnt_ref, wl_ref, wr_ref, b_ref, g_ref, be_ref,
             o_ref):
        agg = jnp.concatenate([agg_ref[0], agg_ref[1]], axis=1)
        h = jnp.concatenate([h_ref[0], h_ref[1]], axis=1)
        cnt = cnt_ref[0] + cnt_ref[1]
        recip = 1.0 / jnp.maximum(cnt, 1.0)
        z = (jnp.dot(agg * recip, wl_ref[...],
                     preferred_element_type=jnp.float32,
                     precision=lax.Precision.HIGHEST)
             + jnp.dot(h, wr_ref[...],
                       preferred_element_type=jnp.float32,
                       precision=lax.Precision.HIGHEST)
             + b_ref[...])
        mu = jnp.mean(z, axis=1, keepdims=True)
        zc = z - mu
        var = jnp.mean(zc * zc, axis=1, keepdims=True)
        hn = zc / jnp.sqrt(var + 1e-5) * g_ref[...] + be_ref[...]
        hn = jnp.maximum(hn, 0.0)
        o_ref[0] = hn[:, :F]
        o_ref[1] = hn[:, F:]

    return pl.pallas_call(
        body,
        grid=(N // R,),
        in_specs=[
            pl.BlockSpec((NC, R, F), lambda i: (0, i, 0)),
            pl.BlockSpec((NC, R, F), lambda i: (0, i, 0)),
            pl.BlockSpec((NC, R, 1), lambda i: (0, i, 0)),
            pl.BlockSpec((H, H), lambda i: (0, 0)),
            pl.BlockSpec((H, H), lambda i: (0, 0)),
            pl.BlockSpec((1, H), lambda i: (0, 0)),
            pl.BlockSpec((1, H), lambda i: (0, 0)),
            pl.BlockSpec((1, H), lambda i: (0, 0)),
        ],
        out_specs=pl.BlockSpec((NC, R, F), lambda i: (0, i, 0)),
        out_shape=jax.ShapeDtypeStruct((NC, N, F), jnp.float32),
    )(agg3, h3, cnt2, wlt, wrt, bl2, gg2, bb2)


def _tc_layer3_pool(agg3, h3, cnt3, wlt, wrt, bl2, gg2, bb2, batch2, gfeat,
                    wm1t, bm12, wm2t, bm22):
    R = 1024
    nblk = N // R

    def body(agg_ref, h_ref, cnt_ref, wl_ref, wr_ref, b_ref, g_ref, be_ref,
             bt_ref, gf_ref, w1_ref, b1_ref, w2_ref, b2_ref, o_ref,
             sum_s, max_s, cnt_s):
        i = pl.program_id(0)

        @pl.when(i == 0)
        def _():
            sum_s[...] = jnp.zeros_like(sum_s)
            max_s[...] = jnp.full_like(max_s, -3.4e38)
            cnt_s[...] = jnp.zeros_like(cnt_s)

        agg = jnp.concatenate([agg_ref[0], agg_ref[1]], axis=1)
        h = jnp.concatenate([h_ref[0], h_ref[1]], axis=1)
        cnt = cnt_ref[0] + cnt_ref[1]
        recip = 1.0 / jnp.maximum(cnt, 1.0)
        z = (jnp.dot(agg * recip, wl_ref[...],
                     preferred_element_type=jnp.float32,
                     precision=lax.Precision.HIGHEST)
             + jnp.dot(h, wr_ref[...],
                       preferred_element_type=jnp.float32,
                       precision=lax.Precision.HIGHEST)
             + b_ref[...])
        mu = jnp.mean(z, axis=1, keepdims=True)
        zc = z - mu
        var = jnp.mean(zc * zc, axis=1, keepdims=True)
        hn = zc / jnp.sqrt(var + 1e-5) * g_ref[...] + be_ref[...]
        hn = jnp.maximum(hn, 0.0)

        b = bt_ref[...]
        glo = jnp.min(b)
        ghi = jnp.max(b)

        def gbody(g, carry):
            mrow = b == g
            csum = jnp.sum(jnp.where(mrow, hn, 0.0), axis=0, keepdims=True)
            cmax = jnp.max(jnp.where(mrow, hn, -3.4e38), axis=0,
                           keepdims=True)
            ccnt = jnp.sum(mrow.astype(jnp.float32))
            sum_s[pl.ds(g, 1), :] += csum
            max_s[pl.ds(g, 1), :] = jnp.maximum(max_s[pl.ds(g, 1), :], cmax)
            cnt_s[pl.ds(g, 1), :] += ccnt
            return carry

        lax.fori_loop(glo, ghi + 1, gbody, 0)

        @pl.when(i == nblk - 1)
        def _():
            pcnt = cnt_s[:, 0:1]
            mean = sum_s[...] / jnp.maximum(pcnt, 1.0)
            mx = jnp.where(pcnt > 0.0, max_s[...], 0.0)
            gcat = jnp.concatenate([mean, mx, gf_ref[...]], axis=1)
            hm = jnp.maximum(
                jnp.dot(gcat, w1_ref[...],
                        preferred_element_type=jnp.float32,
                        precision=lax.Precision.HIGHEST) + b1_ref[...], 0.0)
            o_ref[...] = jnp.dot(hm, w2_ref[...],
                                 preferred_element_type=jnp.float32,
                                 precision=lax.Precision.HIGHEST) + b2_ref[...]

    return pl.pallas_call(
        body,
        grid=(nblk,),
        in_specs=[
            pl.BlockSpec((NC, R, F), lambda i: (0, i, 0)),
            pl.BlockSpec((NC, R, F), lambda i: (0, i, 0)),
            pl.BlockSpec((NC, R, 1), lambda i: (0, i, 0)),
            pl.BlockSpec((H, H), lambda i: (0, 0)),
            pl.BlockSpec((H, H), lambda i: (0, 0)),
            pl.BlockSpec((1, H), lambda i: (0, 0)),
            pl.BlockSpec((1, H), lambda i: (0, 0)),
            pl.BlockSpec((1, H), lambda i: (0, 0)),
            pl.BlockSpec((R, 1), lambda i: (i, 0)),
            pl.BlockSpec((G, GF), lambda i: (0, 0)),
            pl.BlockSpec((2 * H + GF, H), lambda i: (0, 0)),
            pl.BlockSpec((1, H), lambda i: (0, 0)),
            pl.BlockSpec((H, C), lambda i: (0, 0)),
            pl.BlockSpec((1, C), lambda i: (0, 0)),
        ],
        out_specs=pl.BlockSpec((G, C), lambda i: (0, 0)),
        out_shape=jax.ShapeDtypeStruct((G, C), jnp.float32),
        scratch_shapes=[
            pltpu.VMEM((G, H), jnp.float32),
            pltpu.VMEM((G, H), jnp.float32),
            pltpu.VMEM((G, H), jnp.float32),
        ],
    )(agg3, h3, cnt3, wlt, wrt, bl2, gg2, bb2, batch2, gfeat, wm1t, bm12,
      wm2t, bm22)


def kernel(x, edge_index, batch, ptr, root_idx, gfeat, W_l0, b_l0, W_r0, g0,
           be0, W_l1, b_l1, W_r1, g1, be1, W_l2, b_l2, W_r2, g2, be2, Wm1,
           bm1, Wm2, bm2):
    src = edge_index[0]
    dst = edge_index[1]
    src_r = src.reshape(NS * NCHG, CH)
    srcm = jnp.concatenate([src_r, src_r + N], axis=0)
    dstm = dst.reshape(NS * NCHG, CH)

    zeros = jnp.zeros((N, F), jnp.float32)

    cntf = _sc_count(dst)
    cnt3 = cntf.reshape(NC, N, 1)

    h3 = x.reshape(N, NC, F).transpose(1, 0, 2)
    for wl, bl, wr, gg, bb in [(W_l0, b_l0, W_r0, g0, be0),
                               (W_l1, b_l1, W_r1, g1, be1)]:
        aggf = _sc_agg(h3.reshape(NC * N, F), srcm, dstm, zeros)
        h3 = _tc_layer(aggf.reshape(NC, N, F), h3, cnt3, wl.T, wr.T,
                       bl.reshape(1, H), gg.reshape(1, H), bb.reshape(1, H))

    aggf = _sc_agg(h3.reshape(NC * N, F), srcm, dstm, zeros)
    return _tc_layer3_pool(aggf.reshape(NC, N, F), h3, cnt3, W_l2.T, W_r2.T,
                           b_l2.reshape(1, H), g2.reshape(1, H),
                           be2.reshape(1, H), batch.reshape(N, 1), gfeat,
                           Wm1.T, bm1.reshape(1, H), Wm2.T,
                           bm2.reshape(1, C))
